# baseline (speedup 1.0000x reference)
import functools

import jax
import jax.numpy as jnp
from jax import lax
from jax.experimental import pallas as pl
from jax.experimental.pallas import tpu as pltpu
from jax.experimental.pallas import tpu_sc as plsc

_VOCAB = 1000000
_EMB = 64
_B = 4096
_L = 200

_NC = 2
_NS = 16
_NW = _NC * _NS
_BW = _B // _NW
_LANES = 16


@functools.partial(
    pl.kernel,
    mesh=plsc.VectorSubcoreMesh(core_axis_name="c", subcore_axis_name="s"),
    out_type=jax.ShapeDtypeStruct((_L, _EMB, _B), jnp.float32),
    scratch_types=[
        pltpu.VMEM((_L, _BW), jnp.int32),
        pltpu.VMEM((_L, _BW), jnp.int32),
        pltpu.VMEM((_BW, 2 * _EMB), jnp.float32),
        pltpu.VMEM((_BW, 2 * _EMB), jnp.float32),
        pltpu.VMEM((_EMB, _BW), jnp.float32),
        pltpu.VMEM((_EMB, _BW), jnp.float32),
        pltpu.SemaphoreType.DMA,
        pltpu.SemaphoreType.DMA,
        pltpu.SemaphoreType.DMA,
        pltpu.SemaphoreType.DMA,
    ],
    compiler_params=pltpu.CompilerParams(needs_layout_passes=False),
)
def _gather_kernel(tab2_hbm, seqt_hbm, out_hbm,
                   idx_v, off_v, stag0, stag1, slab0, slab1,
                   sg0, sg1, sw0, sw1):
    wid = lax.axis_index("s") * _NC + lax.axis_index("c")
    lane0 = wid * _BW
    stag = (stag0, stag1)
    slab = (slab0, slab1)
    sem_g = (sg0, sg1)
    sem_w = (sw0, sw1)

    def _gather_desc(l, b):
        return pltpu.make_async_copy(
            tab2_hbm.at[idx_v.at[l]],
            stag[b],
            sem_g[b],
        )

    def _write_desc(l, b):
        return pltpu.make_async_copy(
            slab[b],
            out_hbm.at[l, :, pl.ds(lane0, _BW)],
            sem_w[b],
        )

    pltpu.sync_copy(seqt_hbm.at[:, pl.ds(lane0, _BW)], idx_v)

    def _split(l, _):
        for jb in range(_BW // _LANES):
            sl = pl.ds(jb * _LANES, _LANES)
            v = idx_v[l, sl]
            off_v[l, sl] = (v & 1) * _EMB
            idx_v[l, sl] = v >> 1
        return ()

    lax.fori_loop(0, _L, _split, ())

    def _transpose(l, b):
        for jb in range(_BW // _LANES):
            sl = pl.ds(jb * _LANES, _LANES)
            jvec = jb * _LANES + jax.lax.iota(jnp.int32, _LANES)
            offv = off_v[l, sl]

            def _erow(e, _):
                vals = plsc.load_gather(stag[b], [jvec, offv + e])
                slab[b][e, sl] = vals
                return ()

            lax.fori_loop(0, _EMB, _erow, ())

    def _step(l, b, first, last):
        _gather_desc(l, b).wait()
        if not first:
            _write_desc(l - 2, b).wait()
        _transpose(l, b)
        _write_desc(l, b).start()
        if not last:
            _gather_desc(l + 2, b).start()

    _gather_desc(0, 0).start()
    _gather_desc(1, 1).start()

    _step(0, 0, True, False)
    _step(1, 1, True, False)

    def body(p, _):
        for b in range(2):
            _step(2 * p + b, b, False, False)
        return ()

    lax.fori_loop(1, _L // 2 - 1, body, ())

    _step(_L - 2, 0, False, True)
    _step(_L - 1, 1, False, True)

    _write_desc(_L - 2, 0).wait()
    _write_desc(_L - 1, 1).wait()


def kernel(sequences, token_embedding_matrix):
    tab2 = token_embedding_matrix.reshape(_VOCAB // 2, 2 * _EMB)
    seqt = sequences.T.astype(jnp.int32)
    outt = _gather_kernel(tab2, seqt)
    return outt.transpose(2, 0, 1), _L


# speedup vs baseline: 1.4368x; 1.4368x over previous
import functools

import jax
import jax.numpy as jnp
from jax import lax
from jax.experimental import pallas as pl
from jax.experimental.pallas import tpu as pltpu
from jax.experimental.pallas import tpu_sc as plsc

_VOCAB = 1000000
_EMB = 64
_B = 4096
_L = 200

_NC = 2
_NS = 16
_NW = _NC * _NS
_BW = _B // _NW
_LANES = 16


@functools.partial(
    pl.kernel,
    mesh=plsc.VectorSubcoreMesh(core_axis_name="c", subcore_axis_name="s"),
    out_type=jax.ShapeDtypeStruct((_L, _EMB, _B), jnp.float32),
    scratch_types=[
        pltpu.VMEM((_L, _BW), jnp.int32),
        pltpu.VMEM((_L, _BW), jnp.int32),
        pltpu.VMEM((_BW, 2 * _EMB), jnp.float32),
        pltpu.VMEM((_BW, 2 * _EMB), jnp.float32),
        pltpu.VMEM((_EMB, _BW), jnp.float32),
        pltpu.VMEM((_EMB, _BW), jnp.float32),
        pltpu.SemaphoreType.DMA,
        pltpu.SemaphoreType.DMA,
        pltpu.SemaphoreType.DMA,
        pltpu.SemaphoreType.DMA,
    ],
    compiler_params=pltpu.CompilerParams(needs_layout_passes=False),
)
def _gather_kernel(tab2_hbm, seqt_hbm, out_hbm,
                   idx_v, off_v, stag0, stag1, slab0, slab1,
                   sg0, sg1, sw0, sw1):
    wid = lax.axis_index("s") * _NC + lax.axis_index("c")
    lane0 = wid * _BW
    stag = (stag0, stag1)
    slab = (slab0, slab1)
    sem_g = (sg0, sg1)
    sem_w = (sw0, sw1)

    def _gather_desc(l, b):
        return pltpu.make_async_copy(
            tab2_hbm.at[idx_v.at[l]],
            stag[b],
            sem_g[b],
        )

    def _write_desc(l, b):
        return pltpu.make_async_copy(
            slab[b],
            out_hbm.at[l, :, pl.ds(lane0, _BW)],
            sem_w[b],
        )

    pltpu.sync_copy(seqt_hbm.at[:, pl.ds(lane0, _BW)], idx_v)

    def _split(l, _):
        for jb in range(_BW // _LANES):
            sl = pl.ds(jb * _LANES, _LANES)
            v = idx_v[l, sl]
            off_v[l, sl] = (v & 1) * _EMB
            idx_v[l, sl] = v >> 1
        return ()

    lax.fori_loop(0, _L, _split, ())

    def _transpose(l, b):
        for jb in range(_BW // _LANES):
            sl = pl.ds(jb * _LANES, _LANES)
            jvec = jb * _LANES + jax.lax.iota(jnp.int32, _LANES)
            offv = off_v[l, sl]

            @plsc.parallel_loop(0, _EMB, unroll=8)
            def _erow(e):
                vals = plsc.load_gather(stag[b], [jvec, offv + e])
                slab[b][e, sl] = vals

    def _step(l, b, first, last):
        _gather_desc(l, b).wait()
        if not first:
            _write_desc(l - 2, b).wait()
        _transpose(l, b)
        _write_desc(l, b).start()
        if not last:
            _gather_desc(l + 2, b).start()

    _gather_desc(0, 0).start()
    _gather_desc(1, 1).start()

    _step(0, 0, True, False)
    _step(1, 1, True, False)

    def body(p, _):
        for b in range(2):
            _step(2 * p + b, b, False, False)
        return ()

    lax.fori_loop(1, _L // 2 - 1, body, ())

    _step(_L - 2, 0, False, True)
    _step(_L - 1, 1, False, True)

    _write_desc(_L - 2, 0).wait()
    _write_desc(_L - 1, 1).wait()


def kernel(sequences, token_embedding_matrix):
    tab2 = token_embedding_matrix.reshape(_VOCAB // 2, 2 * _EMB)
    seqt = sequences.T.astype(jnp.int32)
    outt = _gather_kernel(tab2, seqt)
    return outt.transpose(2, 0, 1), _L


# speedup vs baseline: 1.4855x; 1.0339x over previous
import functools

import jax
import jax.numpy as jnp
from jax import lax
from jax.experimental import pallas as pl
from jax.experimental.pallas import tpu as pltpu
from jax.experimental.pallas import tpu_sc as plsc

_VOCAB = 1000000
_EMB = 64
_B = 4096
_L = 200
_N = _B * _L

_NC = 2
_NS = 16
_NW = _NC * _NS
_PER_W = _N // _NW

_C = 800
_NCHUNK = _PER_W // _C

assert _PER_W % _C == 0 and _NCHUNK % 2 == 0 and _C % 8 == 0


@functools.partial(
    pl.kernel,
    mesh=plsc.VectorSubcoreMesh(core_axis_name="c", subcore_axis_name="s"),
    out_type=jax.ShapeDtypeStruct((_N, _EMB), jnp.float32),
    scratch_types=[
        pltpu.VMEM((_PER_W,), jnp.int32),
        pltpu.VMEM((2 * _C, _EMB), jnp.float32),
        pltpu.SemaphoreType.DMA,
        pltpu.SemaphoreType.DMA,
        pltpu.SemaphoreType.DMA,
        pltpu.SemaphoreType.DMA,
    ],
    compiler_params=pltpu.CompilerParams(use_tc_tiling_on_sc=False),
)
def _gather_kernel(table_hbm, idx_hbm, out_hbm, idx_v, rows_v, sg0, sg1, sw0, sw1):
    wid = lax.axis_index("s") * _NC + lax.axis_index("c")
    base = wid * _PER_W
    sem_g = (sg0, sg1)
    sem_w = (sw0, sw1)

    def _gather_desc(g, b):
        return pltpu.make_async_copy(
            table_hbm.at[idx_v.at[pl.ds(g * _C, _C)]],
            rows_v.at[pl.ds(b * _C, _C)],
            sem_g[b],
        )

    def _write_desc(g, b):
        return pltpu.make_async_copy(
            rows_v.at[pl.ds(b * _C, _C)],
            out_hbm.at[pl.ds(base + g * _C, _C)],
            sem_w[b],
        )

    pltpu.sync_copy(idx_hbm.at[pl.ds(base, _PER_W)], idx_v)

    _gather_desc(0, 0).start()
    _gather_desc(1, 1).start()

    def body(p, _):
        for b in range(2):
            g = 2 * p + b
            _gather_desc(g - 2, b).wait()
            _write_desc(g - 2, b).start()
            _write_desc(g - 2, b).wait()
            _gather_desc(g, b).start()
        return ()

    lax.fori_loop(1, _NCHUNK // 2, body, ())

    for b in range(2):
        g = _NCHUNK - 2 + b
        _gather_desc(g, b).wait()
        _write_desc(g, b).start()
    for b in range(2):
        g = _NCHUNK - 2 + b
        _write_desc(g, b).wait()


def kernel(sequences, token_embedding_matrix):
    idx = sequences.reshape(_N).astype(jnp.int32)
    flat = _gather_kernel(token_embedding_matrix, idx)
    return flat.reshape(_B, _L, _EMB), _L
